# baseline (speedup 1.0000x reference)
import functools

import jax
import jax.numpy as jnp
from jax import lax
from jax.experimental import pallas as pl
from jax.experimental.pallas import tpu as pltpu
from jax.experimental.pallas import tpu_sc as plsc

DIM = 64
NW = 32
CHUNK = 128
NBUF = 2


def _sc_gather(weight, idx):
    nchunk = idx.shape[1]
    mesh = plsc.VectorSubcoreMesh(core_axis_name="c", subcore_axis_name="s")

    @functools.partial(
        pl.kernel,
        mesh=mesh,
        out_type=jax.ShapeDtypeStruct((NW, nchunk, CHUNK, DIM), jnp.float32),
        scratch_types=[
            pltpu.VMEM((nchunk, CHUNK), jnp.int32),
            pltpu.VMEM((NBUF, CHUNK, DIM), jnp.float32),
            pltpu.SemaphoreType.DMA,
            pltpu.SemaphoreType.DMA,
        ],
        compiler_params=pltpu.CompilerParams(use_tc_tiling_on_sc=False),
    )
    def k(table_hbm, idx_hbm, out_hbm, idx_v, rows_v, sem0, sem1):
        wid = lax.axis_index("s") * 2 + lax.axis_index("c")
        pltpu.sync_copy(idx_hbm.at[wid], idx_v)
        sems = (sem0, sem1)

        def group(g, carry):
            copies = []
            for b in range(NBUF):
                j = g * NBUF + b
                cp = pltpu.async_copy(
                    table_hbm.at[idx_v.at[j]], rows_v.at[b], sems[b]
                )
                copies.append((j, b, cp))
            for j, b, cp in copies:
                cp.wait()
                pltpu.sync_copy(rows_v.at[b], out_hbm.at[wid, j])
            return carry

        lax.fori_loop(0, nchunk // NBUF, group, 0)

    return k(weight, idx)


def kernel(token_ids, weight):
    bsz, seq = token_ids.shape
    total = bsz * seq
    nchunk = total // (NW * CHUNK)
    idx = token_ids.reshape(NW, nchunk, CHUNK).astype(jnp.int32)
    out = _sc_gather(weight, idx)
    return out.reshape(bsz, seq, DIM)


# speedup vs baseline: 1.0403x; 1.0403x over previous
import functools

import jax
import jax.numpy as jnp
from jax import lax
from jax.experimental import pallas as pl
from jax.experimental.pallas import tpu as pltpu
from jax.experimental.pallas import tpu_sc as plsc

DIM = 64
NW = 32
CHUNK = 128
NSUB = 5
NBUF = 2


def _sc_gather(weight, idx):
    nsuper = idx.shape[1] // NSUB
    half = nsuper // NBUF
    mesh = plsc.VectorSubcoreMesh(core_axis_name="c", subcore_axis_name="s")

    @functools.partial(
        pl.kernel,
        mesh=mesh,
        out_type=jax.ShapeDtypeStruct(
            (NW, nsuper, NSUB, CHUNK, DIM), jnp.float32
        ),
        scratch_types=[
            pltpu.VMEM((nsuper * NSUB, CHUNK), jnp.int32),
            pltpu.VMEM((NBUF, NSUB, CHUNK, DIM), jnp.float32),
            pltpu.SemaphoreType.DMA,
            pltpu.SemaphoreType.DMA,
            pltpu.SemaphoreType.DMA,
            pltpu.SemaphoreType.DMA,
        ],
        compiler_params=pltpu.CompilerParams(use_tc_tiling_on_sc=False),
    )
    def k(table_hbm, idx_hbm, out_hbm, idx_v, rows_v, g0, g1, w0, w1):
        wid = lax.axis_index("s") * 2 + lax.axis_index("c")
        pltpu.sync_copy(idx_hbm.at[wid], idx_v)
        gsems = (g0, g1)
        wsems = (w0, w1)

        def fire(buf, s):
            for u in range(NSUB):
                pltpu.async_copy(
                    table_hbm.at[idx_v.at[s * NSUB + u]],
                    rows_v.at[buf, u],
                    gsems[buf],
                )

        def drain_gathers(buf):
            for u in range(NSUB):
                pltpu.make_async_copy(
                    table_hbm.at[idx_v.at[u]], rows_v.at[buf, u], gsems[buf]
                ).wait()

        def start_write(buf, s):
            pltpu.async_copy(rows_v.at[buf], out_hbm.at[wid, s], wsems[buf])

        def drain_write(buf):
            pltpu.make_async_copy(
                rows_v.at[buf], out_hbm.at[wid, 0], wsems[buf]
            ).wait()

        fire(0, 0)
        fire(1, 1)

        def body(h, carry):
            drain_gathers(0)
            start_write(0, NBUF * h)
            drain_gathers(1)
            start_write(1, NBUF * h + 1)

            @pl.when(h < half - 1)
            def _():
                drain_write(0)
                fire(0, NBUF * h + 2)
                drain_write(1)
                fire(1, NBUF * h + 3)

            return carry

        lax.fori_loop(0, half, body, 0)
        drain_write(0)
        drain_write(1)

    return k(weight, idx)


def kernel(token_ids, weight):
    bsz, seq = token_ids.shape
    total = bsz * seq
    nsuper = total // (NW * NSUB * CHUNK)
    idx = token_ids.reshape(NW, nsuper * NSUB, CHUNK).astype(jnp.int32)
    out = _sc_gather(weight, idx)
    return out.reshape(bsz, seq, DIM)


# speedup vs baseline: 1.2689x; 1.2197x over previous
import functools

import jax
import jax.numpy as jnp
from jax import lax
from jax.experimental import pallas as pl
from jax.experimental.pallas import tpu as pltpu
from jax.experimental.pallas import tpu_sc as plsc

PDIM = 128
NW = 32
CHUNK = 128
NSUB = 2
NBUF = 2

NROW = 1000000
NFULL = NROW // CHUNK
TAIL = NROW - NFULL * CHUNK
GROUPS = (NFULL // NW) // NBUF
EXTRA = NFULL - (GROUPS * NBUF) * NW


def _sc_gather(table, idx):
    nsuper = idx.shape[1] // NSUB
    half = nsuper // NBUF
    mesh = plsc.VectorSubcoreMesh(core_axis_name="c", subcore_axis_name="s")

    @functools.partial(
        pl.kernel,
        mesh=mesh,
        out_type=jax.ShapeDtypeStruct(
            (NW, nsuper, NSUB, CHUNK, PDIM), jnp.float32
        ),
        scratch_types=[
            pltpu.VMEM((nsuper * NSUB, CHUNK), jnp.int32),
            pltpu.VMEM((NBUF, NSUB, CHUNK, PDIM), jnp.float32),
            pltpu.SemaphoreType.DMA,
            pltpu.SemaphoreType.DMA,
            pltpu.SemaphoreType.DMA,
            pltpu.SemaphoreType.DMA,
        ],
        compiler_params=pltpu.CompilerParams(use_tc_tiling_on_sc=True),
    )
    def k(table_hbm, idx_hbm, out_hbm, idx_v, rows_v, g0, g1, w0, w1):
        wid = lax.axis_index("s") * 2 + lax.axis_index("c")
        pltpu.sync_copy(idx_hbm.at[wid], idx_v)
        gsems = (g0, g1)
        wsems = (w0, w1)

        def fire(buf, s):
            for u in range(NSUB):
                pltpu.async_copy(
                    table_hbm.at[idx_v.at[s * NSUB + u]],
                    rows_v.at[buf, u],
                    gsems[buf],
                )

        def drain_gathers(buf):
            for u in range(NSUB):
                pltpu.make_async_copy(
                    table_hbm.at[idx_v.at[u]], rows_v.at[buf, u], gsems[buf]
                ).wait()

        def start_write(buf, s):
            pltpu.async_copy(rows_v.at[buf], out_hbm.at[wid, s], wsems[buf])

        def drain_write(buf):
            pltpu.make_async_copy(
                rows_v.at[buf], out_hbm.at[wid, 0], wsems[buf]
            ).wait()

        fire(0, 0)
        fire(1, 1)

        def body(h, carry):
            drain_gathers(0)
            start_write(0, NBUF * h)
            drain_gathers(1)
            start_write(1, NBUF * h + 1)

            @pl.when(h < half - 1)
            def _():
                drain_write(0)
                fire(0, NBUF * h + 2)
                drain_write(1)
                fire(1, NBUF * h + 3)

            return carry

        lax.fori_loop(0, half, body, 0)
        drain_write(0)
        drain_write(1)

    return k(table, idx)


def kernel(token_ids, weight):
    bsz, seq = token_ids.shape
    dim = weight.shape[1]
    total = bsz * seq
    nchunk = total // (NW * CHUNK)
    table = jnp.pad(weight, ((0, 0), (0, PDIM - dim)))
    idx = token_ids.reshape(NW, nchunk, CHUNK).astype(jnp.int32)
    out = _sc_gather(table, idx)
    out = out.reshape(total, PDIM)[:, :dim]
    return out.reshape(bsz, seq, dim)
